# baseline (speedup 1.0000x reference)
import jax, jax.numpy as jnp
from jax import lax
from jax.experimental import pallas as pl
from jax.experimental.pallas import tpu as pltpu
from jax.experimental.pallas import tpu_sc as plsc

VOCAB, HIDDEN, BATCH, SEQ = 100000, 64, 4096, 50
NW = 32; RPW = BATCH // NW
CB = 1
CHUNKS = RPW // CB
LANES = 16; HREGS = HIDDEN // LANES
SCALE = 1.0 / SEQ

def _body(table_hbm, ids_hbm, out_hbm, idx_v, rows0, rows1, out_v, sem0, sem1):
    wid = lax.axis_index("s") * 2 + lax.axis_index("c")
    pltpu.sync_copy(ids_hbm.at[pl.ds(wid * RPW, RPW)], idx_v)

    def start(j, buf, sem):
        pltpu.async_copy(table_hbm.at[idx_v.at[j]], buf, sem)
    def wait(buf, sem):
        pltpu.make_async_copy(table_hbm.at[idx_v.at[0]], buf, sem).wait()
    def accum(j, buf):
        def step(s, acc):
            return tuple(
                acc[r * HREGS + c] + buf[s, pl.ds(c * LANES, LANES)]
                for r in range(CB) for c in range(HREGS))
        zero = jnp.zeros((LANES,), jnp.float32)
        acc = lax.fori_loop(0, SEQ, step, (zero,) * (CB * HREGS), unroll=5)
        for r in range(CB):
            for c in range(HREGS):
                out_v[j * CB + r, pl.ds(c * LANES, LANES)] = acc[r * HREGS + c] * SCALE

    start(0, rows0, sem0)
    def outer(i, _):
        j = 2 * i
        start(j + 1, rows1, sem1)
        wait(rows0, sem0)
        accum(j, rows0)
        start(j + 2, rows0, sem0)
        wait(rows1, sem1)
        accum(j + 1, rows1)
        return 0
    lax.fori_loop(0, CHUNKS // 2 - 1, outer, 0)
    start(CHUNKS - 1, rows1, sem1)
    wait(rows0, sem0)
    accum(CHUNKS - 2, rows0)
    wait(rows1, sem1)
    accum(CHUNKS - 1, rows1)
    pltpu.sync_copy(out_v, out_hbm.at[pl.ds(wid * RPW, RPW)])

@jax.jit
def _run(ids, table):
    mesh = plsc.VectorSubcoreMesh(core_axis_name="c", subcore_axis_name="s")
    f = pl.kernel(
        _body,
        out_type=jax.ShapeDtypeStruct((BATCH, HIDDEN), jnp.float32),
        mesh=mesh,
        scratch_types=[
            pltpu.VMEM((RPW, SEQ), jnp.int32),
            pltpu.VMEM((SEQ, HIDDEN), jnp.float32),
            pltpu.VMEM((SEQ, HIDDEN), jnp.float32),
            pltpu.VMEM((RPW, HIDDEN), jnp.float32),
            pltpu.SemaphoreType.DMA,
            pltpu.SemaphoreType.DMA,
        ],
        compiler_params=pltpu.CompilerParams(use_tc_tiling_on_sc=False),
    )
    return f(table, ids)

def kernel(instruction_ids, embed_table):
    return _run(instruction_ids, embed_table)


# speedup vs baseline: 1.1434x; 1.1434x over previous
import jax, jax.numpy as jnp
from jax import lax
from jax.experimental import pallas as pl
from jax.experimental.pallas import tpu as pltpu
from jax.experimental.pallas import tpu_sc as plsc

VOCAB, HIDDEN, BATCH, SEQ = 100000, 64, 4096, 50
NW = 32; RPW = BATCH // NW
CB = 2
CHUNKS = RPW // CB
LANES = 16; HREGS = HIDDEN // LANES
SCALE = 1.0 / SEQ

def _body(table_hbm, ids_hbm, out_hbm, idx_v, rows0, rows1, out_v, sem0, sem1):
    wid = lax.axis_index("s") * 2 + lax.axis_index("c")
    pltpu.sync_copy(ids_hbm.at[pl.ds(wid * CHUNKS, CHUNKS)], idx_v)

    def start(j, buf, sem):
        pltpu.async_copy(table_hbm.at[idx_v.at[j]], buf, sem)
    def wait(buf, sem):
        pltpu.make_async_copy(table_hbm.at[idx_v.at[0]], buf, sem).wait()
    def accum(j, buf):
        def step(s, acc):
            return tuple(
                acc[r * HREGS + c] + buf[r * SEQ + s, pl.ds(c * LANES, LANES)]
                for r in range(CB) for c in range(HREGS))
        zero = jnp.zeros((LANES,), jnp.float32)
        acc = lax.fori_loop(0, SEQ, step, (zero,) * (CB * HREGS), unroll=5)
        for r in range(CB):
            for c in range(HREGS):
                out_v[j * CB + r, pl.ds(c * LANES, LANES)] = acc[r * HREGS + c] * SCALE

    start(0, rows0, sem0)
    def outer(i, _):
        j = 2 * i
        start(j + 1, rows1, sem1)
        wait(rows0, sem0)
        accum(j, rows0)
        start(j + 2, rows0, sem0)
        wait(rows1, sem1)
        accum(j + 1, rows1)
        return 0
    lax.fori_loop(0, CHUNKS // 2 - 1, outer, 0)
    start(CHUNKS - 1, rows1, sem1)
    wait(rows0, sem0)
    accum(CHUNKS - 2, rows0)
    wait(rows1, sem1)
    accum(CHUNKS - 1, rows1)
    pltpu.sync_copy(out_v, out_hbm.at[pl.ds(wid * RPW, RPW)])

@jax.jit
def _run(ids, table):
    mesh = plsc.VectorSubcoreMesh(core_axis_name="c", subcore_axis_name="s")
    f = pl.kernel(
        _body,
        out_type=jax.ShapeDtypeStruct((BATCH, HIDDEN), jnp.float32),
        mesh=mesh,
        scratch_types=[
            pltpu.VMEM((CHUNKS, CB * SEQ), jnp.int32),
            pltpu.VMEM((CB * SEQ, HIDDEN), jnp.float32),
            pltpu.VMEM((CB * SEQ, HIDDEN), jnp.float32),
            pltpu.VMEM((RPW, HIDDEN), jnp.float32),
            pltpu.SemaphoreType.DMA,
            pltpu.SemaphoreType.DMA,
        ],
        compiler_params=pltpu.CompilerParams(use_tc_tiling_on_sc=False),
    )
    return f(table, ids.reshape(BATCH // CB, CB * SEQ))

def kernel(instruction_ids, embed_table):
    return _run(instruction_ids, embed_table)


# speedup vs baseline: 1.3057x; 1.1419x over previous
import jax, jax.numpy as jnp
from jax import lax
from jax.experimental import pallas as pl
from jax.experimental.pallas import tpu as pltpu
from jax.experimental.pallas import tpu_sc as plsc

VOCAB, HIDDEN, BATCH, SEQ = 100000, 64, 4096, 50
NW = 32; RPW = BATCH // NW
CB = 2
CHUNKS = RPW // CB
NBUF = 4
LANES = 16; HREGS = HIDDEN // LANES
SCALE = 1.0 / SEQ

def _body(table_hbm, ids_hbm, out_hbm, idx_v, r0, r1, r2, r3, out_v,
          s0, s1, s2, s3):
    bufs = (r0, r1, r2, r3)
    sems = (s0, s1, s2, s3)
    wid = lax.axis_index("s") * 2 + lax.axis_index("c")
    pltpu.sync_copy(ids_hbm.at[pl.ds(wid * CHUNKS, CHUNKS)], idx_v)

    def start(j, buf, sem):
        pltpu.async_copy(table_hbm.at[idx_v.at[j]], buf, sem)
    def wait(buf, sem):
        pltpu.make_async_copy(table_hbm.at[idx_v.at[0]], buf, sem).wait()
    def accum(j, buf):
        def step(s, acc):
            return tuple(
                acc[r * HREGS + c] + buf[r * SEQ + s, pl.ds(c * LANES, LANES)]
                for r in range(CB) for c in range(HREGS))
        zero = jnp.zeros((LANES,), jnp.float32)
        acc = lax.fori_loop(0, SEQ, step, (zero,) * (CB * HREGS), unroll=5)
        for r in range(CB):
            for c in range(HREGS):
                out_v[j * CB + r, pl.ds(c * LANES, LANES)] = acc[r * HREGS + c] * SCALE

    for b in range(NBUF - 1):
        start(b, bufs[b], sems[b])
    def outer(i, _):
        base = NBUF * i
        for b in range(NBUF):
            c = base + b
            start(c + NBUF - 1, bufs[(b + NBUF - 1) % NBUF],
                  sems[(b + NBUF - 1) % NBUF])
            wait(bufs[b], sems[b])
            accum(c, bufs[b])
        return 0
    lax.fori_loop(0, CHUNKS // NBUF - 1, outer, 0)
    tail = CHUNKS - NBUF
    start(CHUNKS - 1, bufs[(CHUNKS - 1) % NBUF], sems[(CHUNKS - 1) % NBUF])
    for b in range(NBUF):
        wait(bufs[b], sems[b])
        accum(tail + b, bufs[b])
    pltpu.sync_copy(out_v, out_hbm.at[pl.ds(wid * RPW, RPW)])

@jax.jit
def _run(ids, table):
    mesh = plsc.VectorSubcoreMesh(core_axis_name="c", subcore_axis_name="s")
    f = pl.kernel(
        _body,
        out_type=jax.ShapeDtypeStruct((BATCH, HIDDEN), jnp.float32),
        mesh=mesh,
        scratch_types=[
            pltpu.VMEM((CHUNKS, CB * SEQ), jnp.int32),
            pltpu.VMEM((CB * SEQ, HIDDEN), jnp.float32),
            pltpu.VMEM((CB * SEQ, HIDDEN), jnp.float32),
            pltpu.VMEM((CB * SEQ, HIDDEN), jnp.float32),
            pltpu.VMEM((CB * SEQ, HIDDEN), jnp.float32),
            pltpu.VMEM((RPW, HIDDEN), jnp.float32),
            pltpu.SemaphoreType.DMA,
            pltpu.SemaphoreType.DMA,
            pltpu.SemaphoreType.DMA,
            pltpu.SemaphoreType.DMA,
        ],
        compiler_params=pltpu.CompilerParams(use_tc_tiling_on_sc=False),
    )
    return f(table, ids.reshape(BATCH // CB, CB * SEQ))

def kernel(instruction_ids, embed_table):
    return _run(instruction_ids, embed_table)


# speedup vs baseline: 1.3311x; 1.0194x over previous
import jax, jax.numpy as jnp
from jax import lax
from jax.experimental import pallas as pl
from jax.experimental.pallas import tpu as pltpu
from jax.experimental.pallas import tpu_sc as plsc

VOCAB, HIDDEN, BATCH, SEQ = 100000, 64, 4096, 50
NW = 32; RPW = BATCH // NW
CB = 2
CHUNKS = RPW // CB
NBUF = 8
LANES = 16; HREGS = HIDDEN // LANES
SCALE = 1.0 / SEQ

def _body(table_hbm, ids_hbm, out_hbm, idx_v, r0, r1, r2, r3, r4, r5, r6, r7,
          out_v, s0, s1, s2, s3, s4, s5, s6, s7):
    bufs = (r0, r1, r2, r3, r4, r5, r6, r7)
    sems = (s0, s1, s2, s3, s4, s5, s6, s7)
    wid = lax.axis_index("s") * 2 + lax.axis_index("c")
    pltpu.sync_copy(ids_hbm.at[pl.ds(wid * CHUNKS, CHUNKS)], idx_v)

    def start(j, buf, sem):
        pltpu.async_copy(table_hbm.at[idx_v.at[j]], buf, sem)
    def wait(buf, sem):
        pltpu.make_async_copy(table_hbm.at[idx_v.at[0]], buf, sem).wait()
    def accum(j, buf):
        def step(s, acc):
            return tuple(
                acc[r * HREGS + c] + buf[r * SEQ + s, pl.ds(c * LANES, LANES)]
                for r in range(CB) for c in range(HREGS))
        zero = jnp.zeros((LANES,), jnp.float32)
        acc = lax.fori_loop(0, SEQ, step, (zero,) * (CB * HREGS), unroll=5)
        for r in range(CB):
            for c in range(HREGS):
                out_v[j * CB + r, pl.ds(c * LANES, LANES)] = acc[r * HREGS + c] * SCALE

    for b in range(NBUF - 1):
        start(b, bufs[b], sems[b])
    def outer(i, _):
        base = NBUF * i
        for b in range(NBUF):
            c = base + b
            start(c + NBUF - 1, bufs[(b + NBUF - 1) % NBUF],
                  sems[(b + NBUF - 1) % NBUF])
            wait(bufs[b], sems[b])
            accum(c, bufs[b])
        return 0
    lax.fori_loop(0, CHUNKS // NBUF - 1, outer, 0)
    tail = CHUNKS - NBUF
    start(CHUNKS - 1, bufs[(CHUNKS - 1) % NBUF], sems[(CHUNKS - 1) % NBUF])
    for b in range(NBUF):
        wait(bufs[b], sems[b])
        accum(tail + b, bufs[b])
    pltpu.sync_copy(out_v, out_hbm.at[pl.ds(wid * RPW, RPW)])

@jax.jit
def _run(ids, table):
    mesh = plsc.VectorSubcoreMesh(core_axis_name="c", subcore_axis_name="s")
    f = pl.kernel(
        _body,
        out_type=jax.ShapeDtypeStruct((BATCH, HIDDEN), jnp.float32),
        mesh=mesh,
        scratch_types=[
            pltpu.VMEM((CHUNKS, CB * SEQ), jnp.int32),
        ] + [pltpu.VMEM((CB * SEQ, HIDDEN), jnp.float32)] * NBUF + [
            pltpu.VMEM((RPW, HIDDEN), jnp.float32),
        ] + [pltpu.SemaphoreType.DMA] * NBUF + [
        ],
        compiler_params=pltpu.CompilerParams(use_tc_tiling_on_sc=False),
    )
    return f(table, ids.reshape(BATCH // CB, CB * SEQ))

def kernel(instruction_ids, embed_table):
    return _run(instruction_ids, embed_table)


# speedup vs baseline: 1.3501x; 1.0142x over previous
import jax, jax.numpy as jnp
from jax import lax
from jax.experimental import pallas as pl
from jax.experimental.pallas import tpu as pltpu
from jax.experimental.pallas import tpu_sc as plsc

VOCAB, HIDDEN, BATCH, SEQ = 100000, 64, 4096, 50
NW = 32; RPW = BATCH // NW
CB = 2
CHUNKS = RPW // CB
NBUF = 8
LANES = 16; HREGS = HIDDEN // LANES
SCALE = 1.0 / SEQ

def _body(table_hbm, ids_hbm, out_hbm, idx_v, r0, r1, r2, r3, r4, r5, r6, r7,
          out_v, s0, s1, s2, s3, s4, s5, s6, s7):
    bufs = (r0, r1, r2, r3, r4, r5, r6, r7)
    sems = (s0, s1, s2, s3, s4, s5, s6, s7)
    wid = lax.axis_index("s") * 2 + lax.axis_index("c")
    pltpu.sync_copy(ids_hbm.at[pl.ds(wid * RPW, RPW)], idx_v)

    def start(j, buf, sem):
        for r in range(CB):
            pltpu.async_copy(table_hbm.at[idx_v.at[CB * j + r]],
                             buf.at[pl.ds(r * SEQ, SEQ)], sem)
    def wait(buf, sem):
        for r in range(CB):
            pltpu.make_async_copy(table_hbm.at[idx_v.at[0]],
                                  buf.at[pl.ds(r * SEQ, SEQ)], sem).wait()
    def accum(j, buf):
        def step(s, acc):
            return tuple(
                acc[r * HREGS + c] + buf[r * SEQ + s, pl.ds(c * LANES, LANES)]
                for r in range(CB) for c in range(HREGS))
        zero = jnp.zeros((LANES,), jnp.float32)
        acc = lax.fori_loop(0, SEQ, step, (zero,) * (CB * HREGS), unroll=5)
        for r in range(CB):
            for c in range(HREGS):
                out_v[j * CB + r, pl.ds(c * LANES, LANES)] = acc[r * HREGS + c] * SCALE

    for b in range(NBUF - 1):
        start(b, bufs[b], sems[b])
    def outer(i, _):
        base = NBUF * i
        for b in range(NBUF):
            c = base + b
            start(c + NBUF - 1, bufs[(b + NBUF - 1) % NBUF],
                  sems[(b + NBUF - 1) % NBUF])
            wait(bufs[b], sems[b])
            accum(c, bufs[b])
        return 0
    lax.fori_loop(0, CHUNKS // NBUF - 1, outer, 0)
    tail = CHUNKS - NBUF
    start(CHUNKS - 1, bufs[(CHUNKS - 1) % NBUF], sems[(CHUNKS - 1) % NBUF])
    for b in range(NBUF):
        wait(bufs[b], sems[b])
        accum(tail + b, bufs[b])
    pltpu.sync_copy(out_v, out_hbm.at[pl.ds(wid * RPW, RPW)])

@jax.jit
def _run(ids, table):
    mesh = plsc.VectorSubcoreMesh(core_axis_name="c", subcore_axis_name="s")
    f = pl.kernel(
        _body,
        out_type=jax.ShapeDtypeStruct((BATCH, HIDDEN), jnp.float32),
        mesh=mesh,
        scratch_types=[
            pltpu.VMEM((RPW, SEQ), jnp.int32),
        ] + [pltpu.VMEM((CB * SEQ, HIDDEN), jnp.float32)] * NBUF + [
            pltpu.VMEM((RPW, HIDDEN), jnp.float32),
        ] + [pltpu.SemaphoreType.DMA] * NBUF + [
        ],
        compiler_params=pltpu.CompilerParams(use_tc_tiling_on_sc=False),
    )
    return f(table, ids)

def kernel(instruction_ids, embed_table):
    return _run(instruction_ids, embed_table)
